# baseline (speedup 1.0000x reference)
import jax
import jax.numpy as jnp
from jax import lax
from jax.experimental import pallas as pl

B, S, H = 4, 4096, 1024
EPS = 1e-12
ROWS = B * S
BLK = 2048
NSB = S // BLK


def _tc_body(x_ref, pos_ref, gamma_ref, beta_ref, o_ref):
    x = x_ref[...] + pos_ref[...]
    u = jnp.mean(x, axis=-1, keepdims=True)
    v = jnp.mean(x * x, axis=-1, keepdims=True) - u * u
    inv = lax.rsqrt(v + EPS)
    o_ref[...] = (x - u) * inv * gamma_ref[...] + beta_ref[...]


def kernel(input_ids, position_table, gamma, beta):
    x2 = input_ids.reshape(ROWS, H)
    out = pl.pallas_call(
        _tc_body,
        grid=(NSB, B),
        in_specs=[
            pl.BlockSpec((BLK, H), lambda j, i: (i * NSB + j, 0)),
            pl.BlockSpec((BLK, H), lambda j, i: (j, 0)),
            pl.BlockSpec((1, H), lambda j, i: (0, 0)),
            pl.BlockSpec((1, H), lambda j, i: (0, 0)),
        ],
        out_specs=pl.BlockSpec((BLK, H), lambda j, i: (i * NSB + j, 0)),
        out_shape=jax.ShapeDtypeStruct((ROWS, H), jnp.float32),
    )(x2, position_table, gamma.reshape(1, H), beta.reshape(1, H))
    return out.reshape(B, S, H)
